# baseline (speedup 1.0000x reference)
import functools

import jax
import jax.numpy as jnp
from jax import lax
from jax.experimental import pallas as pl
from jax.experimental.pallas import tpu as pltpu
from jax.experimental.pallas import tpu_sc as plsc

LANES = 16
PANEL = 128


def _make_vegas_kernel(n, d, ninc, num_workers, block_panels):
    npanel = n // PANEL
    per_w = npanel // num_workers
    nblk = per_w // block_panels
    assert nblk % 2 == 0
    block_samples = block_panels * PANEL
    groups = block_panels * (PANEL // LANES)
    stride = 1024
    tbl_sz = d * stride
    ninc_f = float(ninc)
    jac_scale = float(ninc) ** d

    mesh = plsc.VectorSubcoreMesh(core_axis_name="c", subcore_axis_name="s")

    @functools.partial(
        pl.kernel,
        mesh=mesh,
        compiler_params=pltpu.CompilerParams(needs_layout_passes=False),
        out_type=(
            jax.ShapeDtypeStruct((npanel, d, PANEL), jnp.float32),
            jax.ShapeDtypeStruct((n,), jnp.float32),
        ),
        scratch_types=[
            pltpu.VMEM((2 * tbl_sz,), jnp.float32),
            pltpu.VMEM((block_panels, d, PANEL), jnp.float32),
            pltpu.VMEM((block_panels, d, PANEL), jnp.float32),
            pltpu.VMEM((block_panels, d, PANEL), jnp.float32),
            pltpu.VMEM((block_panels, d, PANEL), jnp.float32),
            pltpu.VMEM((block_samples,), jnp.float32),
            pltpu.VMEM((block_samples,), jnp.float32),
            pltpu.SemaphoreType.DMA,
            pltpu.SemaphoreType.DMA,
            pltpu.SemaphoreType.DMA,
            pltpu.SemaphoreType.DMA,
        ],
    )
    def vegas(
        y_h, tbl_h, x_h, jac_h,
        tbl_v, y0, y1, x0, x1, j0, j1, in0, in1, out0, out1,
    ):
        wid = lax.axis_index("s") * 2 + lax.axis_index("c")
        pltpu.sync_copy(tbl_h, tbl_v)
        base = wid * per_w

        def y_slice(b):
            return y_h.at[pl.ds(base + b * block_panels, block_panels)]

        def x_slice(b):
            return x_h.at[pl.ds(base + b * block_panels, block_panels)]

        def jac_slice(b):
            return jac_h.at[pl.ds((base + b * block_panels) * PANEL, block_samples)]

        def compute(y_v, x_v, jac_v):
            @plsc.parallel_loop(0, block_panels, 1, unroll=1)
            def panel_body(pi):
                for sg in range(PANEL // LANES):
                    s = sg * LANES
                    jacv = jnp.full((LANES,), jac_scale, jnp.float32)
                    for dd in range(d):
                        yv = y_v[pi, dd, pl.ds(s, LANES)]
                        t = yv * ninc_f
                        iy = t.astype(jnp.int32)
                        dy = t - iy.astype(jnp.float32)
                        g0 = plsc.load_gather(
                            tbl_v.at[pl.ds(dd * stride, stride)], [iy]
                        )
                        ig = plsc.load_gather(
                            tbl_v.at[pl.ds(tbl_sz + dd * stride, stride)], [iy]
                        )
                        x_v[pi, dd, pl.ds(s, LANES)] = g0 + ig * dy
                        jacv = jacv * ig
                    jac_v[pl.ds(pi * PANEL + s, LANES)] = jacv

        def step(b, ybuf, xbuf, jbuf, isem, osem, first, last):
            pltpu.make_async_copy(y_slice(b), ybuf, isem).wait()
            @pl.when(jnp.logical_not(first))
            def _():
                pltpu.make_async_copy(xbuf, x_slice(b - 2), osem).wait()
                pltpu.make_async_copy(jbuf, jac_slice(b - 2), osem).wait()
            compute(ybuf, xbuf, jbuf)
            @pl.when(jnp.logical_not(last))
            def _():
                pltpu.async_copy(y_slice(b + 2), ybuf, isem)
            pltpu.async_copy(xbuf, x_slice(b), osem)
            pltpu.async_copy(jbuf, jac_slice(b), osem)

        pltpu.async_copy(y_slice(0), y0, in0)
        pltpu.async_copy(y_slice(1), y1, in1)

        def blk_body(k, carry):
            b = k * 2
            step(b, y0, x0, j0, in0, out0, k == 0, k == nblk // 2 - 1)
            step(b + 1, y1, x1, j1, in1, out1, k == 0, k == nblk // 2 - 1)
            return carry

        lax.fori_loop(0, nblk // 2, blk_body, 0, unroll=False)
        for b in (nblk - 2, nblk - 1):
            ybuf, xbuf, jbuf, osem = (y0, x0, j0, out0) if b % 2 == 0 else (y1, x1, j1, out1)
            pltpu.make_async_copy(xbuf, x_slice(b), osem).wait()
            pltpu.make_async_copy(jbuf, jac_slice(b), osem).wait()

    return vegas


def kernel(y, grid, inc):
    n, d = y.shape
    ninc = inc.shape[1]
    y_p = y.reshape(n // PANEL, PANEL, d).transpose(0, 2, 1)
    tbl = jnp.concatenate(
        [
            jnp.pad(grid, ((0, 0), (0, 1024 - (ninc + 1)))),
            jnp.pad(inc, ((0, 0), (0, 1024 - ninc))),
        ]
    ).reshape(-1)
    fn = _make_vegas_kernel(n, d, ninc, num_workers=32, block_panels=16)
    x_p, jac = fn(y_p, tbl)
    x = x_p.transpose(0, 2, 1).reshape(n, d)
    return x, jac


# speedup vs baseline: 1.2796x; 1.2796x over previous
import functools

import jax
import jax.numpy as jnp
from jax import lax
from jax.experimental import pallas as pl
from jax.experimental.pallas import tpu as pltpu
from jax.experimental.pallas import tpu_sc as plsc

LANES = 16
PANEL = 128


def _make_vegas_kernel(n, d, ninc, num_workers, block_panels):
    npanel = n // PANEL
    per_w = npanel // num_workers
    nblk = per_w // block_panels
    assert nblk % 2 == 0
    block_samples = block_panels * PANEL
    groups = block_panels * (PANEL // LANES)
    stride = 1024
    tbl_sz = d * stride
    ninc_f = float(ninc)
    jac_scale = float(ninc) ** d

    mesh = plsc.VectorSubcoreMesh(core_axis_name="c", subcore_axis_name="s")

    @functools.partial(
        pl.kernel,
        mesh=mesh,
        compiler_params=pltpu.CompilerParams(needs_layout_passes=False),
        out_type=(
            jax.ShapeDtypeStruct((npanel, d, PANEL), jnp.float32),
            jax.ShapeDtypeStruct((n,), jnp.float32),
        ),
        scratch_types=[
            pltpu.VMEM((2 * tbl_sz,), jnp.float32),
            pltpu.VMEM((block_panels, d, PANEL), jnp.float32),
            pltpu.VMEM((block_panels, d, PANEL), jnp.float32),
            pltpu.VMEM((block_panels, d, PANEL), jnp.float32),
            pltpu.VMEM((block_panels, d, PANEL), jnp.float32),
            pltpu.VMEM((block_samples,), jnp.float32),
            pltpu.VMEM((block_samples,), jnp.float32),
            pltpu.SemaphoreType.DMA,
            pltpu.SemaphoreType.DMA,
            pltpu.SemaphoreType.DMA,
            pltpu.SemaphoreType.DMA,
        ],
    )
    def vegas(
        y_h, tbl_h, x_h, jac_h,
        tbl_v, y0, y1, x0, x1, j0, j1, in0, in1, out0, out1,
    ):
        wid = lax.axis_index("s") * 2 + lax.axis_index("c")
        pltpu.sync_copy(tbl_h, tbl_v)
        base = wid * per_w

        def y_slice(b):
            return y_h.at[pl.ds(base + b * block_panels, block_panels)]

        def x_slice(b):
            return x_h.at[pl.ds(base + b * block_panels, block_panels)]

        def jac_slice(b):
            return jac_h.at[pl.ds((base + b * block_panels) * PANEL, block_samples)]

        def compute(y_v, x_v, jac_v):
            @plsc.parallel_loop(0, groups, 1, unroll=4)
            def grp_body(g):
                pi = g // (PANEL // LANES)
                s = (g % (PANEL // LANES)) * LANES
                jacv = jnp.full((LANES,), jac_scale, jnp.float32)
                for dd in range(d):
                    yv = y_v[pi, dd, pl.ds(s, LANES)]
                    t = yv * ninc_f
                    iy = t.astype(jnp.int32)
                    dy = t - iy.astype(jnp.float32)
                    g0 = plsc.load_gather(tbl_v.at[pl.ds(dd * stride, stride)], [iy])
                    ig = plsc.load_gather(
                        tbl_v.at[pl.ds(tbl_sz + dd * stride, stride)], [iy]
                    )
                    x_v[pi, dd, pl.ds(s, LANES)] = g0 + ig * dy
                    jacv = jacv * ig
                jac_v[pl.ds(g * LANES, LANES)] = jacv

        def step(b, ybuf, xbuf, jbuf, isem, osem, first, last):
            pltpu.make_async_copy(y_slice(b), ybuf, isem).wait()
            @pl.when(jnp.logical_not(first))
            def _():
                pltpu.make_async_copy(xbuf, x_slice(b - 2), osem).wait()
                pltpu.make_async_copy(jbuf, jac_slice(b - 2), osem).wait()
            compute(ybuf, xbuf, jbuf)
            @pl.when(jnp.logical_not(last))
            def _():
                pltpu.async_copy(y_slice(b + 2), ybuf, isem)
            pltpu.async_copy(xbuf, x_slice(b), osem)
            pltpu.async_copy(jbuf, jac_slice(b), osem)

        pltpu.async_copy(y_slice(0), y0, in0)
        pltpu.async_copy(y_slice(1), y1, in1)

        def blk_body(k, carry):
            b = k * 2
            step(b, y0, x0, j0, in0, out0, k == 0, k == nblk // 2 - 1)
            step(b + 1, y1, x1, j1, in1, out1, k == 0, k == nblk // 2 - 1)
            return carry

        lax.fori_loop(0, nblk // 2, blk_body, 0, unroll=False)
        for b in (nblk - 2, nblk - 1):
            ybuf, xbuf, jbuf, osem = (y0, x0, j0, out0) if b % 2 == 0 else (y1, x1, j1, out1)
            pltpu.make_async_copy(xbuf, x_slice(b), osem).wait()
            pltpu.make_async_copy(jbuf, jac_slice(b), osem).wait()

    return vegas


def kernel(y, grid, inc):
    n, d = y.shape
    ninc = inc.shape[1]
    y_p = y.reshape(n // PANEL, PANEL, d).transpose(0, 2, 1)
    tbl = jnp.concatenate(
        [
            jnp.pad(grid, ((0, 0), (0, 1024 - (ninc + 1)))),
            jnp.pad(inc, ((0, 0), (0, 1024 - ninc))),
        ]
    ).reshape(-1)
    fn = _make_vegas_kernel(n, d, ninc, num_workers=32, block_panels=16)
    x_p, jac = fn(y_p, tbl)
    x = x_p.transpose(0, 2, 1).reshape(n, d)
    return x, jac
